# Initial kernel scaffold; baseline (speedup 1.0000x reference)
import jax
import jax.numpy as jnp
from jax.experimental import pallas as pl


def kernel(gate_logits):
    raise NotImplementedError("write your pallas kernel here")


# speedup vs baseline: 2.2070x; 2.2070x over previous
import functools

import jax
import jax.numpy as jnp
from jax import lax
from jax.experimental import pallas as pl
from jax.experimental.pallas import tpu as pltpu
from jax.experimental.pallas import tpu_sc as plsc

_E = 64
_K = 8
_COEF = 0.01
_NC = 2
_NS = 16
_NW = _NC * _NS
_LANES = 16
_CH = 128
_PAD = _E + 1


def _sc_loss_partials(gl_flat, n_rows):
    rows_per_w = n_rows // _NW
    n_chunks = rows_per_w // _CH
    groups_per_chunk = _CH // _LANES

    mesh = plsc.VectorSubcoreMesh(
        core_axis_name="c", subcore_axis_name="s",
        num_cores=_NC, num_subcores=_NS)

    @functools.partial(
        pl.kernel,
        out_type=jax.ShapeDtypeStruct((_NW, _LANES), jnp.float32),
        mesh=mesh,
        compiler_params=pltpu.CompilerParams(needs_layout_passes=False),
        scratch_types=[
            pltpu.VMEM((_CH * _E,), jnp.float32),
            pltpu.VMEM((_LANES * _PAD,), jnp.float32),
            pltpu.VMEM((_LANES,), jnp.float32),
        ],
    )
    def body(gl_hbm, out_hbm, buf, buft, accv):
        cid = lax.axis_index("c")
        sid = lax.axis_index("s")
        wid = sid * _NC + cid
        row0 = wid * rows_per_w

        iota = lax.iota(jnp.int32, _LANES)
        gbase = iota * _PAD

        def chunk_body(ci, acc):
            base = (row0 + ci * _CH) * _E
            pltpu.sync_copy(gl_hbm.at[pl.ds(base, _CH * _E)], buf)

            def group_body(gi, acc2):
                goff = gi * (_LANES * _E)
                for r in range(_LANES):
                    for gg in range(_E // _LANES):
                        v = buf[pl.ds(goff + r * _E + gg * _LANES, _LANES)]
                        plsc.store_scatter(
                            buft, [iota + (r * _PAD + gg * _LANES)], v)
                neg = jnp.full((_LANES,), -jnp.inf, jnp.float32)
                m = [neg] * _K
                for e in range(_E):
                    x = plsc.load_gather(buft, [gbase + e])
                    for j in range(_K):
                        hi = jnp.maximum(m[j], x)
                        x = jnp.minimum(m[j], x)
                        m[j] = hi
                s = [jnp.exp(mj - m[0]) for mj in m]
                tot = s[0]
                for j in range(1, _K):
                    tot = tot + s[j]
                p = s[0] / tot
                for j in range(1, _K):
                    p = p + s[j] / tot
                return acc2 + p * (1.0 / _K)

            return lax.fori_loop(0, groups_per_chunk, group_body, acc)

        acc0 = jnp.zeros((_LANES,), jnp.float32)
        acc = lax.fori_loop(0, n_chunks, chunk_body, acc0)
        accv[...] = acc
        pltpu.sync_copy(accv, out_hbm.at[wid])

    return body(gl_flat)


def kernel(gate_logits):
    n_rows = gate_logits.size // _E
    gl_flat = gate_logits.reshape(-1)
    parts = _sc_loss_partials(gl_flat, n_rows)
    scale = _COEF * (_E * _E) / (n_rows * _E)
    return jnp.sum(parts) * jnp.float32(scale)


# speedup vs baseline: 2.3595x; 1.0691x over previous
import functools

import jax
import jax.numpy as jnp
from jax import lax
from jax.experimental import pallas as pl
from jax.experimental.pallas import tpu as pltpu
from jax.experimental.pallas import tpu_sc as plsc

_E = 64
_K = 8
_COEF = 0.01
_NC = 2
_NS = 16
_NW = _NC * _NS
_LANES = 16
_CH = 256
_PAD = _E + 1

_SORT8 = [(0, 1), (2, 3), (4, 5), (6, 7),
          (0, 2), (1, 3), (4, 6), (5, 7),
          (1, 2), (5, 6),
          (0, 4), (1, 5), (2, 6), (3, 7),
          (2, 4), (3, 5),
          (1, 2), (3, 4), (5, 6)]
_BITONIC8 = [(0, 4), (1, 5), (2, 6), (3, 7),
             (0, 2), (1, 3), (4, 6), (5, 7),
             (0, 1), (2, 3), (4, 5), (6, 7)]


def _ce(lst, i, j):
    hi = jnp.maximum(lst[i], lst[j])
    lo = jnp.minimum(lst[i], lst[j])
    lst[i], lst[j] = hi, lo


def _merge_top8(a, b):
    m = [jnp.maximum(a[i], b[7 - i]) for i in range(8)]
    for i, j in _BITONIC8:
        _ce(m, i, j)
    return m


def _sc_loss_partials(gl_flat, n_rows):
    rows_per_w = n_rows // _NW
    n_chunks = rows_per_w // _CH
    groups_per_chunk = _CH // _LANES

    mesh = plsc.VectorSubcoreMesh(
        core_axis_name="c", subcore_axis_name="s",
        num_cores=_NC, num_subcores=_NS)

    @functools.partial(
        pl.kernel,
        out_type=jax.ShapeDtypeStruct((_NW, _LANES), jnp.float32),
        mesh=mesh,
        compiler_params=pltpu.CompilerParams(needs_layout_passes=False),
        scratch_types=[
            pltpu.VMEM((_CH * _E,), jnp.float32),
            pltpu.VMEM((_CH * _E,), jnp.float32),
            pltpu.VMEM((_LANES * _PAD,), jnp.float32),
            pltpu.VMEM((_LANES,), jnp.float32),
            pltpu.SemaphoreType.DMA,
            pltpu.SemaphoreType.DMA,
        ],
    )
    def body(gl_hbm, out_hbm, buf_a, buf_b, buft, accv, sem_a, sem_b):
        cid = lax.axis_index("c")
        sid = lax.axis_index("s")
        wid = sid * _NC + cid
        row0 = wid * rows_per_w

        iota = lax.iota(jnp.int32, _LANES)
        gbase = iota * _PAD

        bufs = [buf_a, buf_b]
        sems = [sem_a, sem_b]

        def start_chunk(ci, slot):
            base = (row0 + ci * _CH) * _E
            return pltpu.async_copy(
                gl_hbm.at[pl.ds(base, _CH * _E)], bufs[slot], sems[slot])

        def group_body_for(buf):
            def group_body(gi, acc2):
                goff = gi * (_LANES * _E)
                for r in range(_LANES):
                    for gg in range(_E // _LANES):
                        v = buf[pl.ds(goff + r * _E + gg * _LANES, _LANES)]
                        plsc.store_scatter(
                            buft, [iota + (r * _PAD + gg * _LANES)], v)
                runs = [None, None]
                for g8 in range(_E // _K):
                    sub = [plsc.load_gather(buft, [gbase + (g8 * _K + t)])
                           for t in range(_K)]
                    for i, j in _SORT8:
                        _ce(sub, i, j)
                    c = g8 % 2
                    runs[c] = sub if runs[c] is None else _merge_top8(runs[c], sub)
                m = _merge_top8(runs[0], runs[1])
                s = [jnp.exp(mj - m[0]) for mj in m]
                tot = s[0]
                for j in range(1, _K):
                    tot = tot + s[j]
                p = s[0] / tot
                for j in range(1, _K):
                    p = p + s[j] / tot
                return acc2 + p * (1.0 / _K)

            return group_body

        acc = jnp.zeros((_LANES,), jnp.float32)
        copies = [None, None]
        copies[0] = start_chunk(0, 0)
        for ci in range(n_chunks):
            slot = ci % 2
            if ci + 1 < n_chunks:
                copies[1 - slot] = start_chunk(ci + 1, 1 - slot)
            copies[slot].wait()
            acc = lax.fori_loop(0, groups_per_chunk,
                                group_body_for(bufs[slot]), acc)
        accv[...] = acc
        pltpu.sync_copy(accv, out_hbm.at[wid])

    return body(gl_flat)


def kernel(gate_logits):
    n_rows = gate_logits.size // _E
    gl_flat = gate_logits.reshape(-1)
    parts = _sc_loss_partials(gl_flat, n_rows)
    scale = _COEF * (_E * _E) / (n_rows * _E)
    return jnp.sum(parts) * jnp.float32(scale)


# speedup vs baseline: 2.3610x; 1.0007x over previous
import functools

import jax
import jax.numpy as jnp
from jax import lax
from jax.experimental import pallas as pl
from jax.experimental.pallas import tpu as pltpu
from jax.experimental.pallas import tpu_sc as plsc

_E = 64
_K = 8
_COEF = 0.01
_NC = 2
_NS = 16
_NW = _NC * _NS
_LANES = 16
_CH = 256
_PAD = _E + 1

_SORT8 = [(0, 1), (2, 3), (4, 5), (6, 7),
          (0, 2), (1, 3), (4, 6), (5, 7),
          (1, 2), (5, 6),
          (0, 4), (1, 5), (2, 6), (3, 7),
          (2, 4), (3, 5),
          (1, 2), (3, 4), (5, 6)]
_BITONIC8 = [(0, 4), (1, 5), (2, 6), (3, 7),
             (0, 2), (1, 3), (4, 6), (5, 7),
             (0, 1), (2, 3), (4, 5), (6, 7)]


def _ce(lst, i, j):
    hi = jnp.maximum(lst[i], lst[j])
    lo = jnp.minimum(lst[i], lst[j])
    lst[i], lst[j] = hi, lo


def _merge_top8(a, b):
    m = [jnp.maximum(a[i], b[7 - i]) for i in range(8)]
    for i, j in _BITONIC8:
        _ce(m, i, j)
    return m


def _sc_loss_partials(gl_flat, n_rows):
    rows_per_w = n_rows // _NW
    n_chunks = rows_per_w // _CH
    groups_per_chunk = _CH // _LANES

    mesh = plsc.VectorSubcoreMesh(
        core_axis_name="c", subcore_axis_name="s",
        num_cores=_NC, num_subcores=_NS)

    @functools.partial(
        pl.kernel,
        out_type=jax.ShapeDtypeStruct((_NW, _LANES), jnp.float32),
        mesh=mesh,
        compiler_params=pltpu.CompilerParams(
            needs_layout_passes=False, use_tc_tiling_on_sc=True),
        scratch_types=[
            pltpu.VMEM((_CH * _E,), jnp.float32),
            pltpu.VMEM((_CH * _E,), jnp.float32),
            pltpu.VMEM((_LANES * _PAD,), jnp.float32),
            pltpu.VMEM((_LANES,), jnp.float32),
            pltpu.SemaphoreType.DMA,
            pltpu.SemaphoreType.DMA,
        ],
    )
    def body(gl_hbm, out_hbm, buf_a, buf_b, buft, accv, sem_a, sem_b):
        cid = lax.axis_index("c")
        sid = lax.axis_index("s")
        wid = sid * _NC + cid
        row0 = wid * rows_per_w

        iota = lax.iota(jnp.int32, _LANES)
        gbase = iota * _PAD

        bufs = [buf_a, buf_b]
        sems = [sem_a, sem_b]

        def start_chunk(ci, slot):
            base = (row0 + ci * _CH) * _E
            return pltpu.async_copy(
                gl_hbm.at[pl.ds(base, _CH * _E)], bufs[slot], sems[slot])

        def group_body_for(buf):
            def group_body(gi, acc2):
                goff = gi * (_LANES * _E)
                for r in range(_LANES):
                    for gg in range(_E // _LANES):
                        v = buf[pl.ds(goff + r * _E + gg * _LANES, _LANES)]
                        plsc.store_scatter(
                            buft, [iota + (r * _PAD + gg * _LANES)], v)
                runs = [None, None]
                for g8 in range(_E // _K):
                    sub = [plsc.load_gather(buft, [gbase + (g8 * _K + t)])
                           for t in range(_K)]
                    for i, j in _SORT8:
                        _ce(sub, i, j)
                    c = g8 % 2
                    runs[c] = sub if runs[c] is None else _merge_top8(runs[c], sub)
                m = _merge_top8(runs[0], runs[1])
                s = [jnp.exp(mj - m[0]) for mj in m]
                tot = s[0]
                for j in range(1, _K):
                    tot = tot + s[j]
                p = s[0] / tot
                for j in range(1, _K):
                    p = p + s[j] / tot
                return acc2 + p * (1.0 / _K)

            return group_body

        acc = jnp.zeros((_LANES,), jnp.float32)
        copies = [None, None]
        copies[0] = start_chunk(0, 0)
        for ci in range(n_chunks):
            slot = ci % 2
            if ci + 1 < n_chunks:
                copies[1 - slot] = start_chunk(ci + 1, 1 - slot)
            copies[slot].wait()
            acc = lax.fori_loop(0, groups_per_chunk,
                                group_body_for(bufs[slot]), acc)
        accv[...] = acc
        pltpu.sync_copy(accv, out_hbm.at[wid])

    return body(gl_flat)


def kernel(gate_logits):
    n_rows = gate_logits.size // _E
    gl_flat = gate_logits.reshape(-1)
    parts = _sc_loss_partials(gl_flat, n_rows)
    scale = _COEF * (_E * _E) / (n_rows * _E)
    return jnp.sum(parts) * jnp.float32(scale)


# speedup vs baseline: 2.7852x; 1.1796x over previous
import functools

import jax
import jax.numpy as jnp
from jax import lax
from jax.experimental import pallas as pl
from jax.experimental.pallas import tpu as pltpu
from jax.experimental.pallas import tpu_sc as plsc

_E = 64
_K = 8
_COEF = 0.01
_NC = 2
_NS = 16
_NW = _NC * _NS
_LANES = 16
_CH = 256
_PAD = _E + 1

_SORT8 = [(0, 1), (2, 3), (4, 5), (6, 7),
          (0, 2), (1, 3), (4, 6), (5, 7),
          (1, 2), (5, 6),
          (0, 4), (1, 5), (2, 6), (3, 7),
          (2, 4), (3, 5),
          (1, 2), (3, 4), (5, 6)]
_BITONIC8 = [(0, 4), (1, 5), (2, 6), (3, 7),
             (0, 2), (1, 3), (4, 6), (5, 7),
             (0, 1), (2, 3), (4, 5), (6, 7)]


def _ce(lst, i, j):
    hi = jnp.maximum(lst[i], lst[j])
    lo = jnp.minimum(lst[i], lst[j])
    lst[i], lst[j] = hi, lo


def _merge_top8(a, b):
    m = [jnp.maximum(a[i], b[7 - i]) for i in range(8)]
    for i, j in _BITONIC8:
        _ce(m, i, j)
    return m


def _sc_loss_partials(gl, n_rows):
    slab_rows = gl.shape[1]
    rows_per_w = n_rows // _NW
    w_per_slab = slab_rows // rows_per_w
    n_chunks = rows_per_w // _CH
    groups_per_chunk = _CH // _LANES

    mesh = plsc.VectorSubcoreMesh(
        core_axis_name="c", subcore_axis_name="s",
        num_cores=_NC, num_subcores=_NS)

    @functools.partial(
        pl.kernel,
        out_type=jax.ShapeDtypeStruct((_NW, _LANES), jnp.float32),
        mesh=mesh,
        compiler_params=pltpu.CompilerParams(
            needs_layout_passes=False, use_tc_tiling_on_sc=True),
        scratch_types=[
            pltpu.VMEM((_CH, _E), jnp.float32),
            pltpu.VMEM((_CH, _E), jnp.float32),
            pltpu.VMEM((_LANES * _PAD,), jnp.float32),
            pltpu.VMEM((_LANES,), jnp.float32),
            pltpu.SemaphoreType.DMA,
            pltpu.SemaphoreType.DMA,
        ],
    )
    def body(gl_hbm, out_hbm, buf_a, buf_b, buft, accv, sem_a, sem_b):
        cid = lax.axis_index("c")
        sid = lax.axis_index("s")
        wid = sid * _NC + cid
        slab = wid // w_per_slab
        row0 = (wid % w_per_slab) * rows_per_w

        iota = lax.iota(jnp.int32, _LANES)
        gbase = iota * _PAD

        bufs = [buf_a, buf_b]
        sems = [sem_a, sem_b]

        def start_chunk(ci, slot):
            return pltpu.async_copy(
                gl_hbm.at[slab, pl.ds(row0 + ci * _CH, _CH), :],
                bufs[slot], sems[slot])

        def group_body_for(buf):
            def group_body(gi, acc2):
                goff = gi * _LANES
                for r in range(_LANES):
                    for gg in range(_E // _LANES):
                        v = buf[goff + r, pl.ds(gg * _LANES, _LANES)]
                        plsc.store_scatter(
                            buft, [iota + (r * _PAD + gg * _LANES)], v)
                runs = [None, None]
                for g8 in range(_E // _K):
                    sub = [plsc.load_gather(buft, [gbase + (g8 * _K + t)])
                           for t in range(_K)]
                    for i, j in _SORT8:
                        _ce(sub, i, j)
                    c = g8 % 2
                    runs[c] = sub if runs[c] is None else _merge_top8(runs[c], sub)
                m = _merge_top8(runs[0], runs[1])
                s = [jnp.exp(mj - m[0]) for mj in m]
                tot = s[0]
                for j in range(1, _K):
                    tot = tot + s[j]
                p = s[0] / tot
                for j in range(1, _K):
                    p = p + s[j] / tot
                return acc2 + p * (1.0 / _K)

            return group_body

        acc = jnp.zeros((_LANES,), jnp.float32)
        copies = [None, None]
        copies[0] = start_chunk(0, 0)
        for ci in range(n_chunks):
            slot = ci % 2
            if ci + 1 < n_chunks:
                copies[1 - slot] = start_chunk(ci + 1, 1 - slot)
            copies[slot].wait()
            acc = lax.fori_loop(0, groups_per_chunk,
                                group_body_for(bufs[slot]), acc)
        accv[...] = acc
        pltpu.sync_copy(accv, out_hbm.at[wid])

    return body(gl)


def kernel(gate_logits):
    n_rows = gate_logits.size // _E
    parts = _sc_loss_partials(gate_logits, n_rows)
    scale = _COEF * (_E * _E) / (n_rows * _E)
    return jnp.sum(parts) * jnp.float32(scale)


# speedup vs baseline: 2.8209x; 1.0128x over previous
import functools

import jax
import jax.numpy as jnp
from jax import lax
from jax.experimental import pallas as pl
from jax.experimental.pallas import tpu as pltpu
from jax.experimental.pallas import tpu_sc as plsc

_E = 64
_K = 8
_COEF = 0.01
_NC = 2
_NS = 16
_NW = _NC * _NS
_LANES = 16
_CH = 256
_RPAD = 17
_TILE = _E * _RPAD

_SORT8 = [(0, 1), (2, 3), (4, 5), (6, 7),
          (0, 2), (1, 3), (4, 6), (5, 7),
          (1, 2), (5, 6),
          (0, 4), (1, 5), (2, 6), (3, 7),
          (2, 4), (3, 5),
          (1, 2), (3, 4), (5, 6)]
_BITONIC8 = [(0, 4), (1, 5), (2, 6), (3, 7),
             (0, 2), (1, 3), (4, 6), (5, 7),
             (0, 1), (2, 3), (4, 5), (6, 7)]


def _ce(lst, i, j):
    hi = jnp.maximum(lst[i], lst[j])
    lo = jnp.minimum(lst[i], lst[j])
    lst[i], lst[j] = hi, lo


def _merge_top8(a, b):
    m = [jnp.maximum(a[i], b[7 - i]) for i in range(8)]
    for i, j in _BITONIC8:
        _ce(m, i, j)
    return m


def _sc_loss_partials(gl, n_rows):
    slab_rows = gl.shape[1]
    rows_per_w = n_rows // _NW
    w_per_slab = slab_rows // rows_per_w
    n_chunks = rows_per_w // _CH
    groups_per_chunk = _CH // _LANES

    mesh = plsc.VectorSubcoreMesh(
        core_axis_name="c", subcore_axis_name="s",
        num_cores=_NC, num_subcores=_NS)

    @functools.partial(
        pl.kernel,
        out_type=jax.ShapeDtypeStruct((_NW, _LANES), jnp.float32),
        mesh=mesh,
        compiler_params=pltpu.CompilerParams(needs_layout_passes=False),
        scratch_types=[
            pltpu.VMEM((_CH, _E), jnp.float32),
            pltpu.VMEM((_CH, _E), jnp.float32),
            pltpu.VMEM((2 * _TILE,), jnp.float32),
            pltpu.VMEM((_LANES,), jnp.float32),
            pltpu.SemaphoreType.DMA,
            pltpu.SemaphoreType.DMA,
        ],
    )
    def body(gl_hbm, out_hbm, buf_a, buf_b, buft, accv, sem_a, sem_b):
        cid = lax.axis_index("c")
        sid = lax.axis_index("s")
        wid = sid * _NC + cid
        slab = wid // w_per_slab
        row0 = (wid % w_per_slab) * rows_per_w

        iota = lax.iota(jnp.int32, _LANES)
        s17 = iota * _RPAD

        bufs = [buf_a, buf_b]
        sems = [sem_a, sem_b]

        def start_chunk(ci, slot):
            return pltpu.async_copy(
                gl_hbm.at[slab, pl.ds(row0 + ci * _CH, _CH), :],
                bufs[slot], sems[slot])

        def repack(buf, grow0, toff):
            for r in range(_LANES):
                for gg in range(_E // _LANES):
                    v = buf[grow0 + r, pl.ds(gg * _LANES, _LANES)]
                    plsc.store_scatter(
                        buft, [s17 + (toff + gg * _LANES * _RPAD + r)], v)

        def top8_rp(toff):
            run = None
            for g8 in range(_E // _K):
                sub = [buft[pl.ds(toff + (g8 * _K + t) * _RPAD, _LANES)]
                       for t in range(_K)]
                for i, j in _SORT8:
                    _ce(sub, i, j)
                run = sub if run is None else _merge_top8(run, sub)
            m = run
            s = [jnp.float32(1.0) + jnp.zeros((_LANES,), jnp.float32)] + [
                jnp.exp(mj - m[0]) for mj in m[1:]]
            tot = s[0]
            for j in range(1, _K):
                tot = tot + s[j]
            p = s[0] / tot
            for j in range(1, _K):
                p = p + s[j] / tot
            return p * (1.0 / _K)

        def group_body_for(buf):
            def group_body(gi, acc2):
                repack(buf, gi * 2 * _LANES, 0)
                repack(buf, gi * 2 * _LANES + _LANES, _TILE)
                rp0 = top8_rp(0)
                rp1 = top8_rp(_TILE)
                return acc2 + (rp0 + rp1)

            return group_body

        acc = jnp.zeros((_LANES,), jnp.float32)
        iters_per_chunk = _CH // (2 * _LANES)
        copies = [None, None]
        copies[0] = start_chunk(0, 0)
        for ci in range(n_chunks):
            slot = ci % 2
            if ci + 1 < n_chunks:
                copies[1 - slot] = start_chunk(ci + 1, 1 - slot)
            copies[slot].wait()
            acc = lax.fori_loop(0, iters_per_chunk,
                                group_body_for(bufs[slot]), acc)
        accv[...] = acc
        pltpu.sync_copy(accv, out_hbm.at[wid])

    return body(gl)


def kernel(gate_logits):
    n_rows = gate_logits.size // _E
    parts = _sc_loss_partials(gate_logits, n_rows)
    scale = _COEF * (_E * _E) / (n_rows * _E)
    return jnp.sum(parts) * jnp.float32(scale)
